# baseline (speedup 1.0000x reference)
import functools

import jax
import jax.numpy as jnp
from jax import lax
from jax.experimental import pallas as pl
from jax.experimental.pallas import tpu as pltpu
from jax.experimental.pallas import tpu_sc as plsc

B, P, C = 8, 8192, 150
H = W = 512
N = B * P
NC, NS = 2, 16
NW = NC * NS
PTS_PER_W = N // NW
GATHER_CHUNK = 128
N_CHUNKS = PTS_PER_W // GATHER_CHUNK


def _round_nearest_idx(v):
    c = v * 512.0
    fi = c.astype(jnp.int32)
    tie = fi.astype(jnp.float32) == c
    odd = (fi & 1) == 1
    r = fi - jnp.where(tie & odd, 1, 0)
    return jnp.clip(r, 0, W - 1)


def _sc_gather_labels(xs_flat, ys_flat, labels_2d):
    mesh = plsc.VectorSubcoreMesh(core_axis_name="c", subcore_axis_name="s")

    @functools.partial(
        pl.kernel,
        mesh=mesh,
        out_type=jax.ShapeDtypeStruct((N,), jnp.int32),
        scratch_types=[
            pltpu.VMEM((PTS_PER_W,), jnp.float32),
            pltpu.VMEM((PTS_PER_W,), jnp.float32),
            pltpu.VMEM((N_CHUNKS, GATHER_CHUNK), jnp.int32),
            pltpu.VMEM((PTS_PER_W,), jnp.int32),
            pltpu.SemaphoreType.DMA,
        ],
    )
    def sc_kernel(xs_hbm, ys_hbm, labels_hbm, out_hbm,
                  xv, yv, idx_v, rows_v, sem):
        wid = lax.axis_index("s") * NC + lax.axis_index("c")
        base = wid * PTS_PER_W
        b_base = (wid >> 2) << 18

        pltpu.sync_copy(xs_hbm.at[pl.ds(base, PTS_PER_W)], xv)
        pltpu.sync_copy(ys_hbm.at[pl.ds(base, PTS_PER_W)], yv)

        copies = []
        for j in range(N_CHUNKS):
            idx_row = idx_v.at[j]

            def body(g, _, j=j, idx_row=idx_row):
                ofs = j * GATHER_CHUNK + g * 16
                ix = _round_nearest_idx(xv[pl.ds(ofs, 16)])
                iy = _round_nearest_idx(yv[pl.ds(ofs, 16)])
                idx_row[pl.ds(g * 16, 16)] = b_base + iy * W + ix
                return 0

            lax.fori_loop(0, GATHER_CHUNK // 16, body, 0)
            copies.append(
                pltpu.async_copy(
                    labels_hbm.at[idx_row],
                    rows_v.at[pl.ds(j * GATHER_CHUNK, GATHER_CHUNK)],
                    sem,
                )
            )
        for cp in copies:
            cp.wait()
        pltpu.sync_copy(rows_v, out_hbm.at[pl.ds(base, PTS_PER_W)])

    return sc_kernel(xs_flat, ys_flat, labels_2d)


ROWS_PER_BLK = 2048
N_BLKS = N // ROWS_PER_BLK


def _tc_loss_body(lbl_ref, logits_ref, out_ref):
    i = pl.program_id(0)
    blk = logits_ref[...]
    mx = jnp.max(blk, axis=1, keepdims=True)
    s = jnp.sum(jnp.exp(blk - mx), axis=1, keepdims=True)
    lse = mx + jnp.log(s)
    lbl = lbl_ref[...]
    io = lax.broadcasted_iota(jnp.int32, blk.shape, 1)
    pick = jnp.sum(jnp.where(io == lbl, blk, 0.0), axis=1, keepdims=True)
    part = jnp.sum(lse - pick).reshape(1, 1)
    acc = jnp.where(i == 0, 0.0, out_ref[...]) + part
    out_ref[...] = jnp.where(i == N_BLKS - 1, acc / float(N), acc)


def kernel(logits, coords, labels):
    xs_flat = coords[..., 0].reshape(N)
    ys_flat = coords[..., 1].reshape(N)
    labels_1d = labels.reshape(B * H * W)
    point_labels = _sc_gather_labels(xs_flat, ys_flat, labels_1d)

    logits_2d = logits.reshape(N, C)
    out = pl.pallas_call(
        _tc_loss_body,
        grid=(N_BLKS,),
        in_specs=[
            pl.BlockSpec((ROWS_PER_BLK, 1), lambda i: (i, 0)),
            pl.BlockSpec((ROWS_PER_BLK, C), lambda i: (i, 0)),
        ],
        out_specs=pl.BlockSpec((1, 1), lambda i: (0, 0)),
        out_shape=jax.ShapeDtypeStruct((1, 1), jnp.float32),
    )(point_labels.reshape(N, 1), logits_2d)
    return out[0, 0]
